# baseline (speedup 1.0000x reference)
import functools

import jax
import jax.numpy as jnp
from jax import lax
from jax.experimental import pallas as pl
from jax.experimental.pallas import tpu as pltpu
from jax.experimental.pallas import tpu_sc as plsc

N = 1024
B = 8
GRU = 64
PATCH = 12
NPATCH = 12
OUTW = 12
IB = GRU // 2


def _dot(a, b):
    return jax.lax.dot_general(a.astype(jnp.bfloat16), b.astype(jnp.bfloat16),
                               (((1,), (0,)), ((), ())),
                               preferred_element_type=jnp.float32)


def _dotT(a, b):
    return jax.lax.dot_general(a.astype(jnp.bfloat16), b.astype(jnp.bfloat16),
                               (((0,), (0,)), ((), ())),
                               preferred_element_type=jnp.float32)


def _bdk(W):
    K, H = W.shape
    Wt = jnp.concatenate([W] * B, axis=0)
    Wt = jnp.concatenate([Wt] * B, axis=1)
    ri = jax.lax.broadcasted_iota(jnp.int32, (B * K, B * H), 0) // K
    ci = jax.lax.broadcasted_iota(jnp.int32, (B * K, B * H), 1) // H
    return jnp.where(ri == ci, Wt, jnp.float32(0.0)).astype(jnp.bfloat16)


def _step_body(src_ref, L_ref, wve_ref, w0_ref, w1_ref, wo_ref,
               pe_ref, out_ref, XS_ref, L_s, bdve_s, bd0i_s, bd0s_s,
               bd1i_s, bd1s_s, bdo_s):
    t = pl.program_id(0)
    BG = B * GRU

    @pl.when(t == 0)
    def _():
        XS_ref[...] = jnp.zeros_like(XS_ref)
        L_s[...] = L_ref[...].astype(jnp.bfloat16)
        w0 = w0_ref[...]
        w1 = w1_ref[...]
        bdve_s[...] = _bdk(wve_ref[...])
        bd0i_s[:, :BG] = _bdk(w0[:GRU, :GRU])
        bd0i_s[:, BG:] = _bdk(w0[:GRU, GRU:])
        bd0s_s[:, :BG] = _bdk(w0[GRU:, :GRU])
        bd0s_s[:, BG:] = _bdk(w0[GRU:, GRU:])
        bd1i_s[...] = _bdk(w1[:GRU, :])
        bd1s_s[...] = _bdk(w1[GRU:, :])
        bdo_s[...] = _bdk(wo_ref[...])

    L = L_s[...]
    P = src_ref[...].reshape(B * PATCH, N)
    Xe = _dotT(P, bdve_s[...]) + pe_ref[0]
    XS_ref[:, :BG] = Xe.astype(jnp.bfloat16)
    S = XS_ref[:, BG:].astype(jnp.float32)
    LB = _dot(L, XS_ref[...])
    LXe = LB[:, :BG]
    SA = LB[:, BG:]
    RU = jax.nn.sigmoid(_dot(LXe, bd0i_s[...]) + _dot(SA, bd0s_s[...]))
    r = RU[:, :BG]
    u = RU[:, BG:]
    SB = _dot(L, r * S)
    c = jnp.tanh(_dot(LXe, bd1i_s[...]) + _dot(SB, bd1s_s[...]))
    Snew = u * S + (1.0 - u) * c
    XS_ref[:, BG:] = Snew.astype(jnp.bfloat16)

    @pl.when(t == NPATCH - 1)
    def _():
        out_ref[...] = _dot(Snew, bdo_s[...])


def _bd(W):
    return jnp.kron(jnp.eye(B, dtype=W.dtype), W)


_NWORK = 32
_LANES = 16
_ROWS_PER_W = N // _NWORK
_SLAB = _ROWS_PER_W * N


_NT = 16
_ZCHUNK = N * N // _NT


def _densify_coo(adj_row, adj_col, adj_val):
    nnz = adj_row.shape[0]
    ept = ((nnz + _NT * _LANES - 1) // (_NT * _LANES)) * _LANES
    e_pad = _NT * ept
    pad = e_pad - nnz
    rows = adj_row.astype(jnp.int32)
    cols = adj_col.astype(jnp.int32)
    vals = adj_val
    if pad:
        rows = jnp.concatenate([rows, jnp.zeros((pad,), jnp.int32)])
        cols = jnp.concatenate([cols, jnp.zeros((pad,), jnp.int32)])
        vals = jnp.concatenate([vals, jnp.zeros((pad,), jnp.float32)])
    zchunk = jnp.zeros((_ZCHUNK,), jnp.float32)

    mesh = plsc.VectorSubcoreMesh(core_axis_name="c", subcore_axis_name="s")

    @functools.partial(
        pl.kernel, mesh=mesh,
        out_type=jax.ShapeDtypeStruct((N * N,), jnp.float32),
        scratch_types=[
            pltpu.VMEM((ept,), jnp.int32),
            pltpu.VMEM((ept,), jnp.int32),
            pltpu.VMEM((ept,), jnp.float32),
            pltpu.VMEM((ept,), jnp.int32),
            pltpu.MemorySpace.VMEM_SHARED((N * N,), jnp.float32),
        ],
        compiler_params=pltpu.CompilerParams(needs_layout_passes=False),
    )
    def _scatter(row_hbm, col_hbm, val_hbm, zero_hbm, L_hbm,
                 rv, cv, vv, iv, shared):
        core = lax.axis_index("c")
        sid = lax.axis_index("s")
        on0 = core == 0
        base = sid * ept

        @pl.when(on0)
        def _():
            pltpu.sync_copy(zero_hbm, shared.at[pl.ds(sid * _ZCHUNK, _ZCHUNK)])
            pltpu.sync_copy(row_hbm.at[pl.ds(base, ept)], rv)
            pltpu.sync_copy(col_hbm.at[pl.ds(base, ept)], cv)
            pltpu.sync_copy(val_hbm.at[pl.ds(base, ept)], vv)

            def ebody(i, carry):
                sl = pl.ds(i * _LANES, _LANES)
                iv[sl] = rv[sl] * N + cv[sl]
                return carry

            lax.fori_loop(0, ept // _LANES, ebody, 0)

        plsc.subcore_barrier()

        @pl.when(on0)
        def _():
            pltpu.sync_copy(vv, shared.at[iv], add=True)

        plsc.subcore_barrier()

        @pl.when(on0)
        def _():
            sl = pl.ds(sid * _ZCHUNK, _ZCHUNK)
            pltpu.sync_copy(shared.at[sl], L_hbm.at[sl])

    return _scatter(rows, cols, vals, zchunk).reshape(N, N)


def kernel(source, select_dataset, W_ve, pe, W0, b0, W1, b1, W_out, b_out,
           adj_row, adj_col, adj_val):
    f32 = jnp.float32

    bf16 = jnp.bfloat16

    Ld = _densify_coo(adj_row, adj_col, adj_val)

    src4 = jnp.squeeze(source, -1).astype(bf16).reshape(B, NPATCH, PATCH, N)

    W_out_pad = jnp.concatenate(
        [W_out, jnp.zeros((GRU - IB, OUTW), f32)], axis=0)
    petile = jnp.tile(pe[0, :NPATCH], (1, B)).reshape(NPATCH, 1, B * GRU)

    BG = B * GRU
    full = lambda shape: pl.BlockSpec(shape, lambda t: tuple(0 for _ in shape))
    out = pl.pallas_call(
        _step_body,
        grid=(NPATCH,),
        in_specs=[
            pl.BlockSpec((B, 1, PATCH, N), lambda t: (0, t, 0, 0)),
            full((N, N)),
            full((PATCH, GRU)),
            full((2 * GRU, 2 * GRU)),
            full((2 * GRU, GRU)),
            full((GRU, OUTW)),
            pl.BlockSpec((1, 1, BG), lambda t: (t, 0, 0)),
        ],
        out_specs=pl.BlockSpec((N, B * OUTW), lambda t: (0, 0)),
        out_shape=jax.ShapeDtypeStruct((N, B * OUTW), f32),
        scratch_shapes=[
            pltpu.VMEM((N, 2 * BG), jnp.bfloat16),
            pltpu.VMEM((N, N), jnp.bfloat16),
            pltpu.VMEM((B * PATCH, BG), jnp.bfloat16),
            pltpu.VMEM((BG, 2 * BG), jnp.bfloat16),
            pltpu.VMEM((BG, 2 * BG), jnp.bfloat16),
            pltpu.VMEM((BG, BG), jnp.bfloat16),
            pltpu.VMEM((BG, BG), jnp.bfloat16),
            pltpu.VMEM((BG, B * OUTW), jnp.bfloat16),
        ],
        compiler_params=pltpu.CompilerParams(
            dimension_semantics=("arbitrary",)),
    )(src4, Ld, W_ve, W0, W1, W_out_pad, petile)

    return jnp.transpose(out.reshape(N, B, OUTW), (1, 2, 0))[..., None]


# speedup vs baseline: 1.0066x; 1.0066x over previous
import functools

import jax
import jax.numpy as jnp
from jax import lax
from jax.experimental import pallas as pl
from jax.experimental.pallas import tpu as pltpu
from jax.experimental.pallas import tpu_sc as plsc

N = 1024
B = 8
GRU = 64
PATCH = 12
NPATCH = 12
OUTW = 12
IB = GRU // 2


def _dot(a, b):
    return jax.lax.dot_general(a.astype(jnp.bfloat16), b.astype(jnp.bfloat16),
                               (((1,), (0,)), ((), ())),
                               preferred_element_type=jnp.float32)


def _dotT(a, b):
    return jax.lax.dot_general(a.astype(jnp.bfloat16), b.astype(jnp.bfloat16),
                               (((0,), (0,)), ((), ())),
                               preferred_element_type=jnp.float32)


def _bdk(W):
    K, H = W.shape
    Wt = jnp.concatenate([W] * B, axis=0)
    Wt = jnp.concatenate([Wt] * B, axis=1)
    ri = jax.lax.broadcasted_iota(jnp.int32, (B * K, B * H), 0) // K
    ci = jax.lax.broadcasted_iota(jnp.int32, (B * K, B * H), 1) // H
    return jnp.where(ri == ci, Wt, jnp.float32(0.0)).astype(jnp.bfloat16)


def _step_body(src_ref, L_ref, wve_ref, w0_ref, w1_ref, wo_ref,
               pe_ref, out_ref, XS_ref, bdve_s, bd0i_s, bd0s_s,
               bd1i_s, bd1s_s, bdo_s):
    t = pl.program_id(0)
    BG = B * GRU

    @pl.when(t == 0)
    def _():
        XS_ref[...] = jnp.zeros_like(XS_ref)
        w0 = w0_ref[...]
        w1 = w1_ref[...]
        bdve_s[...] = _bdk(wve_ref[...])
        bd0i_s[:, :BG] = _bdk(w0[:GRU, :GRU])
        bd0i_s[:, BG:] = _bdk(w0[:GRU, GRU:])
        bd0s_s[:, :BG] = _bdk(w0[GRU:, :GRU])
        bd0s_s[:, BG:] = _bdk(w0[GRU:, GRU:])
        bd1i_s[...] = _bdk(w1[:GRU, :])
        bd1s_s[...] = _bdk(w1[GRU:, :])
        bdo_s[...] = _bdk(wo_ref[...])

    L = L_ref[...]
    P = src_ref[...].reshape(B * PATCH, N)
    Xe = _dotT(P, bdve_s[...]) + pe_ref[0]
    XS_ref[:, :BG] = Xe.astype(jnp.bfloat16)
    S = XS_ref[:, BG:].astype(jnp.float32)
    LB = _dot(L, XS_ref[...])
    LXe = LB[:, :BG]
    SA = LB[:, BG:]
    RU = jax.nn.sigmoid(_dot(LXe, bd0i_s[...]) + _dot(SA, bd0s_s[...]))
    r = RU[:, :BG]
    u = RU[:, BG:]
    SB = _dot(L, r * S)
    c = jnp.tanh(_dot(LXe, bd1i_s[...]) + _dot(SB, bd1s_s[...]))
    Snew = u * S + (1.0 - u) * c
    XS_ref[:, BG:] = Snew.astype(jnp.bfloat16)

    @pl.when(t == NPATCH - 1)
    def _():
        out_ref[...] = _dot(Snew, bdo_s[...])


def _bd(W):
    return jnp.kron(jnp.eye(B, dtype=W.dtype), W)


_NWORK = 32
_LANES = 16
_ROWS_PER_W = N // _NWORK
_SLAB = _ROWS_PER_W * N


_NT = 16
_ZCHUNK = N * N // _NT


def _densify_coo(adj_row, adj_col, adj_val):
    nnz = adj_row.shape[0]
    ept = ((nnz + _NT * _LANES - 1) // (_NT * _LANES)) * _LANES
    e_pad = _NT * ept
    pad = e_pad - nnz
    rows = adj_row.astype(jnp.int32)
    cols = adj_col.astype(jnp.int32)
    vals = adj_val
    if pad:
        rows = jnp.concatenate([rows, jnp.zeros((pad,), jnp.int32)])
        cols = jnp.concatenate([cols, jnp.zeros((pad,), jnp.int32)])
        vals = jnp.concatenate([vals, jnp.zeros((pad,), jnp.float32)])
    zchunk = jnp.zeros((_ZCHUNK,), jnp.float32)

    mesh = plsc.VectorSubcoreMesh(core_axis_name="c", subcore_axis_name="s")

    @functools.partial(
        pl.kernel, mesh=mesh,
        out_type=jax.ShapeDtypeStruct((N * N,), jnp.float32),
        scratch_types=[
            pltpu.VMEM((ept,), jnp.int32),
            pltpu.VMEM((ept,), jnp.int32),
            pltpu.VMEM((ept,), jnp.float32),
            pltpu.VMEM((ept,), jnp.int32),
            pltpu.MemorySpace.VMEM_SHARED((N * N,), jnp.float32),
        ],
        compiler_params=pltpu.CompilerParams(needs_layout_passes=False),
    )
    def _scatter(row_hbm, col_hbm, val_hbm, zero_hbm, L_hbm,
                 rv, cv, vv, iv, shared):
        core = lax.axis_index("c")
        sid = lax.axis_index("s")
        on0 = core == 0
        base = sid * ept

        @pl.when(on0)
        def _():
            pltpu.sync_copy(zero_hbm, shared.at[pl.ds(sid * _ZCHUNK, _ZCHUNK)])
            pltpu.sync_copy(row_hbm.at[pl.ds(base, ept)], rv)
            pltpu.sync_copy(col_hbm.at[pl.ds(base, ept)], cv)
            pltpu.sync_copy(val_hbm.at[pl.ds(base, ept)], vv)

            def ebody(i, carry):
                sl = pl.ds(i * _LANES, _LANES)
                iv[sl] = rv[sl] * N + cv[sl]
                return carry

            lax.fori_loop(0, ept // _LANES, ebody, 0)

        plsc.subcore_barrier()

        @pl.when(on0)
        def _():
            pltpu.sync_copy(vv, shared.at[iv], add=True)

        plsc.subcore_barrier()

        @pl.when(on0)
        def _():
            sl = pl.ds(sid * _ZCHUNK, _ZCHUNK)
            pltpu.sync_copy(shared.at[sl], L_hbm.at[sl])

    return _scatter(rows, cols, vals, zchunk).reshape(N, N)


def kernel(source, select_dataset, W_ve, pe, W0, b0, W1, b1, W_out, b_out,
           adj_row, adj_col, adj_val):
    f32 = jnp.float32

    bf16 = jnp.bfloat16

    Ld = _densify_coo(adj_row, adj_col, adj_val).astype(bf16)

    src4 = jnp.squeeze(source, -1).astype(bf16).reshape(B, NPATCH, PATCH, N)

    W_out_pad = jnp.concatenate(
        [W_out, jnp.zeros((GRU - IB, OUTW), f32)], axis=0)
    petile = jnp.tile(pe[0, :NPATCH], (1, B)).reshape(NPATCH, 1, B * GRU)

    BG = B * GRU
    full = lambda shape: pl.BlockSpec(shape, lambda t: tuple(0 for _ in shape))
    out = pl.pallas_call(
        _step_body,
        grid=(NPATCH,),
        in_specs=[
            pl.BlockSpec((B, 1, PATCH, N), lambda t: (0, t, 0, 0)),
            full((N, N)),
            full((PATCH, GRU)),
            full((2 * GRU, 2 * GRU)),
            full((2 * GRU, GRU)),
            full((GRU, OUTW)),
            pl.BlockSpec((1, 1, BG), lambda t: (t, 0, 0)),
        ],
        out_specs=pl.BlockSpec((N, B * OUTW), lambda t: (0, 0)),
        out_shape=jax.ShapeDtypeStruct((N, B * OUTW), f32),
        scratch_shapes=[
            pltpu.VMEM((N, 2 * BG), jnp.bfloat16),
            pltpu.VMEM((B * PATCH, BG), jnp.bfloat16),
            pltpu.VMEM((BG, 2 * BG), jnp.bfloat16),
            pltpu.VMEM((BG, 2 * BG), jnp.bfloat16),
            pltpu.VMEM((BG, BG), jnp.bfloat16),
            pltpu.VMEM((BG, BG), jnp.bfloat16),
            pltpu.VMEM((BG, B * OUTW), jnp.bfloat16),
        ],
    )(src4, Ld, W_ve, W0, W1, W_out_pad, petile)

    return jnp.transpose(out.reshape(N, B, OUTW), (1, 2, 0))[..., None]


# speedup vs baseline: 1.0102x; 1.0036x over previous
import functools

import jax
import jax.numpy as jnp
from jax import lax
from jax.experimental import pallas as pl
from jax.experimental.pallas import tpu as pltpu
from jax.experimental.pallas import tpu_sc as plsc

N = 1024
B = 8
GRU = 64
PATCH = 12
NPATCH = 12
OUTW = 12
IB = GRU // 2


def _dot(a, b):
    return jax.lax.dot_general(a.astype(jnp.bfloat16), b.astype(jnp.bfloat16),
                               (((1,), (0,)), ((), ())),
                               preferred_element_type=jnp.float32)


def _dotT(a, b):
    return jax.lax.dot_general(a.astype(jnp.bfloat16), b.astype(jnp.bfloat16),
                               (((0,), (0,)), ((), ())),
                               preferred_element_type=jnp.float32)


def _bdk(W):
    K, H = W.shape
    Wt = jnp.concatenate([W] * B, axis=0)
    Wt = jnp.concatenate([Wt] * B, axis=1)
    ri = jax.lax.broadcasted_iota(jnp.int32, (B * K, B * H), 0) // K
    ci = jax.lax.broadcasted_iota(jnp.int32, (B * K, B * H), 1) // H
    return jnp.where(ri == ci, Wt, jnp.float32(0.0)).astype(jnp.bfloat16)


def _step_body(src_ref, L_ref, wve_ref, w0_ref, w1_ref, wo_ref,
               pe_ref, out_ref, XS_ref, bdve_s, bd0i_s, bd0s_s,
               bd1i_s, bd1s_s, bdo_s):
    t = pl.program_id(0)
    BG = B * GRU

    @pl.when(t == 0)
    def _():
        XS_ref[...] = jnp.zeros_like(XS_ref)
        w0 = w0_ref[...]
        w1 = w1_ref[...]
        bdve_s[...] = _bdk(wve_ref[...])
        bd0i_s[:, :BG] = _bdk(w0[:GRU, :GRU])
        bd0i_s[:, BG:] = _bdk(w0[:GRU, GRU:])
        bd0s_s[:, :BG] = _bdk(w0[GRU:, :GRU])
        bd0s_s[:, BG:] = _bdk(w0[GRU:, GRU:])
        bd1i_s[...] = _bdk(w1[:GRU, :])
        bd1s_s[...] = _bdk(w1[GRU:, :])
        bdo_s[...] = _bdk(wo_ref[...])

    L = L_ref[...]
    P = src_ref[...].reshape(B * PATCH, N)
    Xe = _dotT(P, bdve_s[...]) + pe_ref[0]
    XS_ref[:, :BG] = Xe.astype(jnp.bfloat16)
    S = XS_ref[:, BG:].astype(jnp.float32)
    LB = _dot(L, XS_ref[...])
    LXe = LB[:, :BG]
    SA = LB[:, BG:]
    RU = jax.nn.sigmoid(_dot(LXe, bd0i_s[...]) + _dot(SA, bd0s_s[...]))
    r = RU[:, :BG]
    u = RU[:, BG:]
    SB = _dot(L, r * S)
    c = jnp.tanh(_dot(LXe, bd1i_s[...]) + _dot(SB, bd1s_s[...]))
    Snew = u * S + (1.0 - u) * c
    XS_ref[:, BG:] = Snew.astype(jnp.bfloat16)

    @pl.when(t == NPATCH - 1)
    def _():
        out_ref[...] = _dot(Snew, bdo_s[...])


_LANES = 16
_NT = 16
_ZCHUNK = N * N // _NT


def _densify_coo(adj_row, adj_col, adj_val):
    nnz = adj_row.shape[0]
    ept = ((nnz + _NT * _LANES - 1) // (_NT * _LANES)) * _LANES
    e_pad = _NT * ept
    pad = e_pad - nnz
    rows = adj_row.astype(jnp.int32)
    cols = adj_col.astype(jnp.int32)
    vals = adj_val
    if pad:
        rows = jnp.concatenate([rows, jnp.zeros((pad,), jnp.int32)])
        cols = jnp.concatenate([cols, jnp.zeros((pad,), jnp.int32)])
        vals = jnp.concatenate([vals, jnp.zeros((pad,), jnp.float32)])
    zchunk = jnp.zeros((_ZCHUNK,), jnp.float32)

    mesh = plsc.VectorSubcoreMesh(core_axis_name="c", subcore_axis_name="s")

    @functools.partial(
        pl.kernel, mesh=mesh,
        out_type=jax.ShapeDtypeStruct((N * N,), jnp.float32),
        scratch_types=[
            pltpu.VMEM((ept,), jnp.int32),
            pltpu.VMEM((ept,), jnp.int32),
            pltpu.VMEM((ept,), jnp.float32),
            pltpu.VMEM((ept,), jnp.int32),
            pltpu.MemorySpace.VMEM_SHARED((N * N,), jnp.float32),
        ],
        compiler_params=pltpu.CompilerParams(needs_layout_passes=False),
    )
    def _scatter(row_hbm, col_hbm, val_hbm, zero_hbm, L_hbm,
                 rv, cv, vv, iv, shared):
        core = lax.axis_index("c")
        sid = lax.axis_index("s")
        on0 = core == 0
        base = sid * ept

        @pl.when(on0)
        def _():
            pltpu.sync_copy(zero_hbm, shared.at[pl.ds(sid * _ZCHUNK, _ZCHUNK)])
            pltpu.sync_copy(row_hbm.at[pl.ds(base, ept)], rv)
            pltpu.sync_copy(col_hbm.at[pl.ds(base, ept)], cv)
            pltpu.sync_copy(val_hbm.at[pl.ds(base, ept)], vv)

            def ebody(i, carry):
                sl = pl.ds(i * _LANES, _LANES)
                iv[sl] = rv[sl] * N + cv[sl]
                return carry

            lax.fori_loop(0, ept // _LANES, ebody, 0)

        plsc.subcore_barrier()

        @pl.when(on0)
        def _():
            pltpu.sync_copy(vv, shared.at[iv], add=True)

        plsc.subcore_barrier()

        @pl.when(on0)
        def _():
            sl = pl.ds(sid * _ZCHUNK, _ZCHUNK)
            pltpu.sync_copy(shared.at[sl], L_hbm.at[sl])

    return _scatter(rows, cols, vals, zchunk).reshape(N, N)


def kernel(source, select_dataset, W_ve, pe, W0, b0, W1, b1, W_out, b_out,
           adj_row, adj_col, adj_val):
    f32 = jnp.float32

    bf16 = jnp.bfloat16

    Ld = _densify_coo(adj_row, adj_col, adj_val).astype(bf16)

    src4 = jnp.squeeze(source, -1).astype(bf16).reshape(B, NPATCH, PATCH, N)

    W_out_pad = jnp.concatenate(
        [W_out, jnp.zeros((GRU - IB, OUTW), f32)], axis=0)
    petile = jnp.tile(pe[0, :NPATCH], (1, B)).reshape(NPATCH, 1, B * GRU)

    BG = B * GRU
    full = lambda shape: pl.BlockSpec(shape, lambda t: tuple(0 for _ in shape))
    out = pl.pallas_call(
        _step_body,
        grid=(NPATCH,),
        in_specs=[
            pl.BlockSpec((B, 1, PATCH, N), lambda t: (0, t, 0, 0)),
            full((N, N)),
            full((PATCH, GRU)),
            full((2 * GRU, 2 * GRU)),
            full((2 * GRU, GRU)),
            full((GRU, OUTW)),
            pl.BlockSpec((1, 1, BG), lambda t: (t, 0, 0)),
        ],
        out_specs=pl.BlockSpec((N, B * OUTW), lambda t: (0, 0)),
        out_shape=jax.ShapeDtypeStruct((N, B * OUTW), f32),
        scratch_shapes=[
            pltpu.VMEM((N, 2 * BG), jnp.bfloat16),
            pltpu.VMEM((B * PATCH, BG), jnp.bfloat16),
            pltpu.VMEM((BG, 2 * BG), jnp.bfloat16),
            pltpu.VMEM((BG, 2 * BG), jnp.bfloat16),
            pltpu.VMEM((BG, BG), jnp.bfloat16),
            pltpu.VMEM((BG, BG), jnp.bfloat16),
            pltpu.VMEM((BG, B * OUTW), jnp.bfloat16),
        ],
    )(src4, Ld, W_ve, W0, W1, W_out_pad, petile)

    return jnp.transpose(out.reshape(N, B, OUTW), (1, 2, 0))[..., None]
